# Initial kernel scaffold; baseline (speedup 1.0000x reference)
import jax
import jax.numpy as jnp
from jax.experimental import pallas as pl


def kernel(features_list, seqs, type_emb, node_type, edge_index, fc_W, fc_b, hade_W, hade_b, proto_W, proto_b, hfin_W, hfin_b, gcn_W, gcn_b, re_W, re_b, re_wt, gt_Wl, gt_Wr, gt_al, gt_ar, gt_Wrs, gt_Wrt, gt_Wf, gt_ln_g, gt_ln_b, pred_W, pred_b):
    raise NotImplementedError("write your pallas kernel here")


# speedup vs baseline: 1.8720x; 1.8720x over previous
import functools

import jax
import jax.numpy as jnp
from jax import lax
from jax.experimental import pallas as pl
from jax.experimental.pallas import tpu as pltpu
from jax.experimental.pallas import tpu_sc as plsc

N = 10000
E = 320000
D = 128
T = 4
NHEADS = 4
HEAD_DIM = 32
RL_DIM = 128
NUM_GNNS = 4
NUM_GT = 2
B = 1024
L = 16
C = 7

NS = 16
KE = 80
BN = 1000


def _sc_segsum(table, gidx5, sidx5, zeros, np_rows, dt):
    nch = gidx5.shape[2]
    cb = gidx5.shape[3]
    rpt = np_rows // NS

    mesh = plsc.VectorSubcoreMesh(core_axis_name="c", subcore_axis_name="s")

    @functools.partial(
        pl.kernel,
        mesh=mesh,
        out_type=jax.ShapeDtypeStruct((2, np_rows, dt), jnp.float32),
        scratch_types=[
            pltpu.VMEM_SHARED((np_rows, dt), jnp.float32),
            pltpu.VMEM((cb, KE), jnp.int32),
            pltpu.VMEM((cb, KE), jnp.int32),
            pltpu.VMEM((KE, dt), jnp.float32),
            pltpu.SemaphoreType.DMA,
        ],
    )
    def k(table_h, gidx_h, sidx_h, zeros_h, out_h, acc, gi, si, rows, sem):
        c = lax.axis_index("c")
        s = lax.axis_index("s")
        pltpu.sync_copy(zeros_h.at[pl.ds(s * rpt, rpt)],
                        acc.at[pl.ds(s * rpt, rpt)])
        plsc.subcore_barrier()

        def chunk(t, carry):
            pltpu.sync_copy(gidx_h.at[c, s, t], gi)
            pltpu.sync_copy(sidx_h.at[c, s, t], si)

            def body(j, cc):
                pltpu.async_copy(table_h.at[c].at[gi.at[j]], rows, sem).wait()
                pltpu.sync_copy(rows, acc.at[si.at[j]], add=True)
                return cc

            return lax.fori_loop(0, cb, body, carry)

        lax.fori_loop(0, nch, chunk, 0)
        plsc.subcore_barrier()
        pltpu.sync_copy(acc.at[pl.ds(s * rpt, rpt)],
                        out_h.at[c, pl.ds(s * rpt, rpt)])

    return k(table, gidx5, sidx5, zeros)


def _sc_gather(table, idx3, n_out, dt):
    nbt = idx3.shape[1]
    kb = idx3.shape[2]

    mesh = plsc.VectorSubcoreMesh(core_axis_name="c", subcore_axis_name="s")

    @functools.partial(
        pl.kernel,
        mesh=mesh,
        out_type=jax.ShapeDtypeStruct((2, n_out, dt), jnp.float32),
        scratch_types=[
            pltpu.VMEM((nbt, kb), jnp.int32),
            pltpu.VMEM((kb, dt), jnp.float32),
            pltpu.SemaphoreType.DMA,
        ],
    )
    def k(table_h, idx_h, out_h, gi, rows, sem):
        c = lax.axis_index("c")
        s = lax.axis_index("s")
        pltpu.sync_copy(idx_h.at[s], gi)

        def body(j, carry):
            pltpu.async_copy(table_h.at[c].at[gi.at[j]], rows, sem).wait()
            pltpu.sync_copy(rows, out_h.at[c, pl.ds((s * nbt + j) * kb, kb)])
            return carry

        lax.fori_loop(0, nbt, body, 0)

    return k(table, idx3)


def _init_body(nt_ref, te_ref, hw_ref, hb_ref, pw_ref, pb_ref, rwt_ref,
               q_ref, wt_ref):
    nt = nt_ref[...]
    r0 = nt @ te_ref[...]
    z = jnp.maximum(r0 @ hw_ref[...] + hb_ref[...], 0.0)
    sc = z @ pw_ref[...] + pb_ref[...]
    sc = sc - jnp.max(sc, axis=-1, keepdims=True)
    ex = jnp.exp(sc)
    p = ex / jnp.sum(ex, axis=-1, keepdims=True)
    q_ref[:, 0:T] = p
    q_ref[:, T:T + 1] = jnp.ones((BN, 1), jnp.float32)
    q_ref[:, T + 1:] = jnp.zeros((BN, D - T - 1), jnp.float32)
    wt_ref[...] = nt @ rwt_ref[...]


def _proj_body(x_ref, w_ref, b_ref, o_ref):
    o_ref[0] = x_ref[0] @ w_ref[0] + b_ref[0]


def _postdeg_body(deg_ref, hfw_ref, hfb_ref, gh_ref, rew_ref, wt_ref,
                  iso_ref, isi_ref, r_ref, tab_ref):
    d0 = deg_ref[0]
    d1 = deg_ref[1]
    in_deg = jnp.maximum(d0[:, T:T + 1], 1.0)
    out_deg = jnp.maximum(d1[:, T:T + 1], 1.0)
    isi = lax.rsqrt(in_deg)
    iso = lax.rsqrt(out_deg)
    isi_ref[...] = isi
    iso_ref[...] = iso
    n_dist = d0[:, 0:T] / in_deg
    r = jnp.maximum(n_dist @ hfw_ref[...] + hfb_ref[...], 0.0)
    r_ref[...] = r
    tab_ref[0] = gh_ref[...] * iso
    tab_ref[1] = ((r * iso) @ rew_ref[...]) * wt_ref[...]


def _layer_body(agg_ref, iso_ref, isi_ref, gw_ref, gb_ref, rb_ref,
                rewn_ref, wtn_ref, gh_ref, r_ref, tab_ref):
    isi = isi_ref[...]
    iso = iso_ref[...]
    gh = jnp.maximum((agg_ref[0] @ gw_ref[...]) * isi + gb_ref[...], 0.0)
    r = jnp.maximum(agg_ref[1] * isi + rb_ref[...], 0.0)
    gh_ref[...] = gh
    r_ref[...] = r
    tab_ref[0] = gh * iso
    tab_ref[1] = ((r * iso) @ rewn_ref[...]) * wtn_ref[...]


def _agt_body(h_ref, rh_ref, wl_ref, wr_ref, al_ref, ar_ref, wrs_ref,
              wrt_ref, wf_ref, g_ref, b_ref, o_ref, bs):
    x = h_ref[...]
    rh = rh_ref[...]
    fl = x @ wl_ref[...]
    fr = x @ wr_ref[...]
    rk = rh @ wrs_ref[...]
    rq = rh @ wrt_ref[...]
    flk = jnp.where(fl > 0, fl, 0.01 * fl)
    frk = jnp.where(fr > 0, fr, 0.01 * fr)
    al = al_ref[...]
    ar = ar_ref[...]
    dn_rc = (((1,), (1,)), ((), ()))
    row_blocks = []
    for b in range(bs):
        rs = slice(b * L, (b + 1) * L)
        col_blocks = []
        for h in range(NHEADS):
            hc = slice(h * HEAD_DIM, (h + 1) * HEAD_DIM)
            rc = slice(h * RL_DIM, (h + 1) * RL_DIM)
            sl = lax.dot_general(flk[rs, hc], al, dn_rc)
            sr = lax.dot_general(ar, frk[rs, hc], dn_rc)
            s2 = lax.dot_general(rk[rs, rc], rq[rs, rc], dn_rc)
            sc = sl + sr + s2
            sc = sc - jnp.max(sc, axis=-1, keepdims=True)
            ex = jnp.exp(sc)
            sm = ex / jnp.sum(ex, axis=-1, keepdims=True)
            col_blocks.append(sm @ fr[rs, hc])
        row_blocks.append(jnp.concatenate(col_blocks, axis=1))
    ctx = jnp.concatenate(row_blocks, axis=0)
    xo = x + ctx @ wf_ref[...]
    mu = jnp.mean(xo, axis=-1, keepdims=True)
    xc = xo - mu
    var = jnp.mean(xc * xc, axis=-1, keepdims=True)
    o_ref[...] = xc * lax.rsqrt(var + 1e-5) * g_ref[...] + b_ref[...]


def _pred_body(x_ref, w_ref, b_ref, o_ref):
    o_ref[...] = x_ref[...] @ w_ref[...] + b_ref[...]


def kernel(features_list, seqs, type_emb, node_type, edge_index, fc_W, fc_b,
           hade_W, hade_b, proto_W, proto_b, hfin_W, hfin_b, gcn_W, gcn_b,
           re_W, re_b, re_wt, gt_Wl, gt_Wr, gt_al, gt_ar, gt_Wrs, gt_Wrt,
           gt_Wf, gt_ln_g, gt_ln_b, pred_W, pred_b):
    f32 = jnp.float32
    src = edge_index[0].astype(jnp.int32)
    dst = edge_index[1].astype(jnp.int32)
    NP = 10240
    src3 = src.reshape(NS, 10, E // NS // KE // 10, KE)
    dst3 = dst.reshape(NS, 10, E // NS // KE // 10, KE)
    gidx = jnp.stack([src3, src3])
    sidx_dd = jnp.stack([dst3, dst3])
    sidx_ds = jnp.stack([dst3, src3])
    zeros_d = jnp.zeros((NP, D), f32)
    nt1h = jax.nn.one_hot(node_type, T, dtype=f32)

    grid_n = N // BN
    full = lambda shp: pl.BlockSpec(shp, lambda i: tuple(0 for _ in shp))
    rowblk = lambda w: pl.BlockSpec((BN, w), lambda i: (i, 0))
    rowblk2 = lambda w: pl.BlockSpec((2, BN, w), lambda i: (0, i, 0))

    q_tab, wtmap = pl.pallas_call(
        _init_body,
        grid=(grid_n,),
        in_specs=[rowblk(T), full((T, T)), full((T, D)), full((1, D)),
                  full((D, T)), full((1, T)), full((T, NUM_GNNS))],
        out_specs=[rowblk(D), rowblk(NUM_GNNS)],
        out_shape=[jax.ShapeDtypeStruct((N, D), f32),
                   jax.ShapeDtypeStruct((N, NUM_GNNS), f32)],
    )(nt1h, type_emb, hade_W, hade_b.reshape(1, D), proto_W,
      proto_b.reshape(1, T), re_wt.T)

    gh0 = pl.pallas_call(
        _proj_body,
        grid=(T,),
        in_specs=[pl.BlockSpec((1, N // T, D), lambda t: (t, 0, 0)),
                  pl.BlockSpec((1, D, D), lambda t: (t, 0, 0)),
                  pl.BlockSpec((1, 1, D), lambda t: (t, 0, 0))],
        out_specs=pl.BlockSpec((1, N // T, D), lambda t: (t, 0, 0)),
        out_shape=jax.ShapeDtypeStruct((T, N // T, D), f32),
    )(features_list, fc_W, fc_b.reshape(T, 1, D)).reshape(N, D)

    deg = _sc_segsum(jnp.stack([q_tab, q_tab]), gidx, sidx_ds, zeros_d, NP, D)

    iso, isi, r, tab = pl.pallas_call(
        _postdeg_body,
        grid=(grid_n,),
        in_specs=[rowblk2(D), full((T, D)), full((1, D)), rowblk(D),
                  full((D, D)), rowblk(1)],
        out_specs=[rowblk(1), rowblk(1), rowblk(D), rowblk2(D)],
        out_shape=[jax.ShapeDtypeStruct((N, 1), f32),
                   jax.ShapeDtypeStruct((N, 1), f32),
                   jax.ShapeDtypeStruct((N, D), f32),
                   jax.ShapeDtypeStruct((2, N, D), f32)],
    )(deg, hfin_W, hfin_b.reshape(1, D), gh0, re_W[0], wtmap[:, 0:1])

    gh = gh0
    for l in range(NUM_GNNS):
        agg = _sc_segsum(tab, gidx, sidx_dd, zeros_d, NP, D)
        ln = min(l + 1, NUM_GNNS - 1)
        gh, r, tab = pl.pallas_call(
            _layer_body,
            grid=(grid_n,),
            in_specs=[rowblk2(D), rowblk(1), rowblk(1), full((D, D)),
                      full((1, D)), full((1, D)), full((D, D)), rowblk(1)],
            out_specs=[rowblk(D), rowblk(D), rowblk2(D)],
            out_shape=[jax.ShapeDtypeStruct((N, D), f32),
                       jax.ShapeDtypeStruct((N, D), f32),
                       jax.ShapeDtypeStruct((2, N, D), f32)],
        )(agg, iso, isi, gcn_W[l], gcn_b[l].reshape(1, D),
          re_b[l].reshape(1, D), re_W[ln], wtmap[:, ln:ln + 1])

    kb = 64
    idx3 = seqs.reshape(NS, B * L // kb // NS, kb).astype(jnp.int32)
    seq_hr = _sc_gather(jnp.stack([gh, r]), idx3, B * L, D)
    h_seq = seq_hr[0]
    r_seq = seq_hr[1]

    bs = 8
    grid_b = B // bs
    seqblk = pl.BlockSpec((bs * L, D), lambda i: (i, 0))
    for l in range(NUM_GT):
        h_seq = pl.pallas_call(
            functools.partial(_agt_body, bs=bs),
            grid=(grid_b,),
            in_specs=[seqblk, seqblk, full((D, D)), full((D, D)),
                      full((1, HEAD_DIM)), full((1, HEAD_DIM)),
                      full((D, RL_DIM * NHEADS)), full((D, RL_DIM * NHEADS)),
                      full((D, D)), full((1, D)), full((1, D))],
            out_specs=seqblk,
            out_shape=jax.ShapeDtypeStruct((B * L, D), f32),
        )(h_seq, r_seq, gt_Wl[l], gt_Wr[l], gt_al[l].reshape(1, HEAD_DIM),
          gt_ar[l].reshape(1, HEAD_DIM), gt_Wrs[l], gt_Wrt[l], gt_Wf[l],
          gt_ln_g[l].reshape(1, D), gt_ln_b[l].reshape(1, D))

    x0 = h_seq.reshape(B, L, D)[:, 0, :]
    out = pl.pallas_call(
        _pred_body,
        in_specs=[pl.BlockSpec((B, D), lambda: (0, 0)),
                  pl.BlockSpec((D, C), lambda: (0, 0)),
                  pl.BlockSpec((1, C), lambda: (0, 0))],
        out_specs=pl.BlockSpec((B, C), lambda: (0, 0)),
        out_shape=jax.ShapeDtypeStruct((B, C), f32),
    )(x0, pred_W, pred_b.reshape(1, C))
    return out


# speedup vs baseline: 2.2636x; 1.2092x over previous
import functools

import jax
import jax.numpy as jnp
from jax import lax
from jax.experimental import pallas as pl
from jax.experimental.pallas import tpu as pltpu
from jax.experimental.pallas import tpu_sc as plsc

N = 10000
E = 320000
D = 128
T = 4
NHEADS = 4
HEAD_DIM = 32
RL_DIM = 128
NUM_GNNS = 4
NUM_GT = 2
B = 1024
L = 16
C = 7

NS = 16
KE = 80
BN = 1000


def _sc_segsum(table, gidx5, sidx5, zeros, np_rows, dt):
    nch = gidx5.shape[2]
    cb = gidx5.shape[3]
    rpt = np_rows // NS

    mesh = plsc.VectorSubcoreMesh(core_axis_name="c", subcore_axis_name="s")

    @functools.partial(
        pl.kernel,
        mesh=mesh,
        out_type=jax.ShapeDtypeStruct((2, np_rows, dt), jnp.float32),
        scratch_types=[
            pltpu.VMEM_SHARED((np_rows, dt), jnp.float32),
            pltpu.VMEM((cb, KE), jnp.int32),
            pltpu.VMEM((cb, KE), jnp.int32),
            pltpu.VMEM((KE, dt), jnp.float32),
            pltpu.VMEM((KE, dt), jnp.float32),
            pltpu.SemaphoreType.DMA,
            pltpu.SemaphoreType.DMA,
        ],
    )
    def k(table_h, gidx_h, sidx_h, zeros_h, out_h, acc, gi, si,
          rows0, rows1, sem0, sem1):
        c = lax.axis_index("c")
        s = lax.axis_index("s")
        rbuf = (rows0, rows1)
        sems = (sem0, sem1)
        pltpu.sync_copy(zeros_h.at[pl.ds(s * rpt, rpt)],
                        acc.at[pl.ds(s * rpt, rpt)])
        plsc.subcore_barrier()

        def chunk(t, carry):
            pltpu.sync_copy(gidx_h.at[c, s, t], gi)
            pltpu.sync_copy(sidx_h.at[c, s, t], si)
            pend = pltpu.async_copy(table_h.at[c].at[gi.at[0]], rbuf[0],
                                    sems[0])
            for j in range(cb):
                nxt = None
                if j + 1 < cb:
                    nxt = pltpu.async_copy(table_h.at[c].at[gi.at[j + 1]],
                                           rbuf[(j + 1) % 2],
                                           sems[(j + 1) % 2])
                pend.wait()
                pltpu.sync_copy(rbuf[j % 2], acc.at[si.at[j]], add=True)
                pend = nxt
            return carry

        lax.fori_loop(0, nch, chunk, 0)
        plsc.subcore_barrier()
        pltpu.sync_copy(acc.at[pl.ds(s * rpt, rpt)],
                        out_h.at[c, pl.ds(s * rpt, rpt)])

    return k(table, gidx5, sidx5, zeros)


def _sc_gather(table, idx3, n_out, dt):
    nbt = idx3.shape[1]
    kb = idx3.shape[2]

    mesh = plsc.VectorSubcoreMesh(core_axis_name="c", subcore_axis_name="s")

    @functools.partial(
        pl.kernel,
        mesh=mesh,
        out_type=jax.ShapeDtypeStruct((2, n_out, dt), jnp.float32),
        scratch_types=[
            pltpu.VMEM((nbt, kb), jnp.int32),
            pltpu.VMEM((kb, dt), jnp.float32),
            pltpu.SemaphoreType.DMA,
        ],
    )
    def k(table_h, idx_h, out_h, gi, rows, sem):
        c = lax.axis_index("c")
        s = lax.axis_index("s")
        pltpu.sync_copy(idx_h.at[s], gi)

        def body(j, carry):
            pltpu.async_copy(table_h.at[c].at[gi.at[j]], rows, sem).wait()
            pltpu.sync_copy(rows, out_h.at[c, pl.ds((s * nbt + j) * kb, kb)])
            return carry

        lax.fori_loop(0, nbt, body, 0)

    return k(table, idx3)


def _init_body(nt_ref, te_ref, hw_ref, hb_ref, pw_ref, pb_ref, rwt_ref,
               q_ref, wt_ref):
    nt = nt_ref[...]
    r0 = nt @ te_ref[...]
    z = jnp.maximum(r0 @ hw_ref[...] + hb_ref[...], 0.0)
    sc = z @ pw_ref[...] + pb_ref[...]
    sc = sc - jnp.max(sc, axis=-1, keepdims=True)
    ex = jnp.exp(sc)
    p = ex / jnp.sum(ex, axis=-1, keepdims=True)
    q_ref[:, 0:T] = p
    q_ref[:, T:T + 1] = jnp.ones((BN, 1), jnp.float32)
    q_ref[:, T + 1:] = jnp.zeros((BN, D - T - 1), jnp.float32)
    wt_ref[...] = nt @ rwt_ref[...]


def _proj_body(x_ref, w_ref, b_ref, o_ref):
    o_ref[0] = x_ref[0] @ w_ref[0] + b_ref[0]


def _postdeg_body(deg_ref, hfw_ref, hfb_ref, gh_ref, rew_ref, wt_ref,
                  iso_ref, isi_ref, r_ref, tab_ref):
    d0 = deg_ref[0]
    d1 = deg_ref[1]
    in_deg = jnp.maximum(d0[:, T:T + 1], 1.0)
    out_deg = jnp.maximum(d1[:, T:T + 1], 1.0)
    isi = lax.rsqrt(in_deg)
    iso = lax.rsqrt(out_deg)
    isi_ref[...] = isi
    iso_ref[...] = iso
    n_dist = d0[:, 0:T] / in_deg
    r = jnp.maximum(n_dist @ hfw_ref[...] + hfb_ref[...], 0.0)
    r_ref[...] = r
    tab_ref[0] = gh_ref[...] * iso
    tab_ref[1] = ((r * iso) @ rew_ref[...]) * wt_ref[...]


def _layer_body(agg_ref, iso_ref, isi_ref, gw_ref, gb_ref, rb_ref,
                rewn_ref, wtn_ref, gh_ref, r_ref, tab_ref):
    isi = isi_ref[...]
    iso = iso_ref[...]
    gh = jnp.maximum((agg_ref[0] @ gw_ref[...]) * isi + gb_ref[...], 0.0)
    r = jnp.maximum(agg_ref[1] * isi + rb_ref[...], 0.0)
    gh_ref[...] = gh
    r_ref[...] = r
    tab_ref[0] = gh * iso
    tab_ref[1] = ((r * iso) @ rewn_ref[...]) * wtn_ref[...]


def _agt_body(h_ref, rh_ref, wl_ref, wr_ref, al_ref, ar_ref, wrs_ref,
              wrt_ref, wf_ref, g_ref, b_ref, o_ref, bs):
    x = h_ref[...]
    rh = rh_ref[...]
    fl = x @ wl_ref[...]
    fr = x @ wr_ref[...]
    rk = rh @ wrs_ref[...]
    rq = rh @ wrt_ref[...]
    flk = jnp.where(fl > 0, fl, 0.01 * fl)
    frk = jnp.where(fr > 0, fr, 0.01 * fr)
    al = al_ref[...]
    ar = ar_ref[...]
    dn_rc = (((1,), (1,)), ((), ()))
    row_blocks = []
    for b in range(bs):
        rs = slice(b * L, (b + 1) * L)
        col_blocks = []
        for h in range(NHEADS):
            hc = slice(h * HEAD_DIM, (h + 1) * HEAD_DIM)
            rc = slice(h * RL_DIM, (h + 1) * RL_DIM)
            sl = lax.dot_general(flk[rs, hc], al, dn_rc)
            sr = lax.dot_general(ar, frk[rs, hc], dn_rc)
            s2 = lax.dot_general(rk[rs, rc], rq[rs, rc], dn_rc)
            sc = sl + sr + s2
            sc = sc - jnp.max(sc, axis=-1, keepdims=True)
            ex = jnp.exp(sc)
            sm = ex / jnp.sum(ex, axis=-1, keepdims=True)
            col_blocks.append(sm @ fr[rs, hc])
        row_blocks.append(jnp.concatenate(col_blocks, axis=1))
    ctx = jnp.concatenate(row_blocks, axis=0)
    xo = x + ctx @ wf_ref[...]
    mu = jnp.mean(xo, axis=-1, keepdims=True)
    xc = xo - mu
    var = jnp.mean(xc * xc, axis=-1, keepdims=True)
    o_ref[...] = xc * lax.rsqrt(var + 1e-5) * g_ref[...] + b_ref[...]


def _pred_body(x_ref, w_ref, b_ref, o_ref):
    o_ref[...] = x_ref[...] @ w_ref[...] + b_ref[...]


def kernel(features_list, seqs, type_emb, node_type, edge_index, fc_W, fc_b,
           hade_W, hade_b, proto_W, proto_b, hfin_W, hfin_b, gcn_W, gcn_b,
           re_W, re_b, re_wt, gt_Wl, gt_Wr, gt_al, gt_ar, gt_Wrs, gt_Wrt,
           gt_Wf, gt_ln_g, gt_ln_b, pred_W, pred_b):
    f32 = jnp.float32
    src = edge_index[0].astype(jnp.int32)
    dst = edge_index[1].astype(jnp.int32)
    NP = 10240
    src3 = src.reshape(NS, 10, E // NS // KE // 10, KE)
    dst3 = dst.reshape(NS, 10, E // NS // KE // 10, KE)
    gidx = jnp.stack([src3, src3])
    sidx_dd = jnp.stack([dst3, dst3])
    sidx_ds = jnp.stack([dst3, src3])
    zeros_d = jnp.zeros((NP, D), f32)
    nt1h = jax.nn.one_hot(node_type, T, dtype=f32)

    grid_n = N // BN
    full = lambda shp: pl.BlockSpec(shp, lambda i: tuple(0 for _ in shp))
    rowblk = lambda w: pl.BlockSpec((BN, w), lambda i: (i, 0))
    rowblk2 = lambda w: pl.BlockSpec((2, BN, w), lambda i: (0, i, 0))

    q_tab, wtmap = pl.pallas_call(
        _init_body,
        grid=(grid_n,),
        in_specs=[rowblk(T), full((T, T)), full((T, D)), full((1, D)),
                  full((D, T)), full((1, T)), full((T, NUM_GNNS))],
        out_specs=[rowblk(D), rowblk(NUM_GNNS)],
        out_shape=[jax.ShapeDtypeStruct((N, D), f32),
                   jax.ShapeDtypeStruct((N, NUM_GNNS), f32)],
    )(nt1h, type_emb, hade_W, hade_b.reshape(1, D), proto_W,
      proto_b.reshape(1, T), re_wt.T)

    gh0 = pl.pallas_call(
        _proj_body,
        grid=(T,),
        in_specs=[pl.BlockSpec((1, N // T, D), lambda t: (t, 0, 0)),
                  pl.BlockSpec((1, D, D), lambda t: (t, 0, 0)),
                  pl.BlockSpec((1, 1, D), lambda t: (t, 0, 0))],
        out_specs=pl.BlockSpec((1, N // T, D), lambda t: (t, 0, 0)),
        out_shape=jax.ShapeDtypeStruct((T, N // T, D), f32),
    )(features_list, fc_W, fc_b.reshape(T, 1, D)).reshape(N, D)

    deg = _sc_segsum(jnp.stack([q_tab, q_tab]), gidx, sidx_ds, zeros_d, NP, D)

    iso, isi, r, tab = pl.pallas_call(
        _postdeg_body,
        grid=(grid_n,),
        in_specs=[rowblk2(D), full((T, D)), full((1, D)), rowblk(D),
                  full((D, D)), rowblk(1)],
        out_specs=[rowblk(1), rowblk(1), rowblk(D), rowblk2(D)],
        out_shape=[jax.ShapeDtypeStruct((N, 1), f32),
                   jax.ShapeDtypeStruct((N, 1), f32),
                   jax.ShapeDtypeStruct((N, D), f32),
                   jax.ShapeDtypeStruct((2, N, D), f32)],
    )(deg, hfin_W, hfin_b.reshape(1, D), gh0, re_W[0], wtmap[:, 0:1])

    gh = gh0
    for l in range(NUM_GNNS):
        agg = _sc_segsum(tab, gidx, sidx_dd, zeros_d, NP, D)
        ln = min(l + 1, NUM_GNNS - 1)
        gh, r, tab = pl.pallas_call(
            _layer_body,
            grid=(grid_n,),
            in_specs=[rowblk2(D), rowblk(1), rowblk(1), full((D, D)),
                      full((1, D)), full((1, D)), full((D, D)), rowblk(1)],
            out_specs=[rowblk(D), rowblk(D), rowblk2(D)],
            out_shape=[jax.ShapeDtypeStruct((N, D), f32),
                       jax.ShapeDtypeStruct((N, D), f32),
                       jax.ShapeDtypeStruct((2, N, D), f32)],
        )(agg, iso, isi, gcn_W[l], gcn_b[l].reshape(1, D),
          re_b[l].reshape(1, D), re_W[ln], wtmap[:, ln:ln + 1])

    kb = 64
    idx3 = seqs.reshape(NS, B * L // kb // NS, kb).astype(jnp.int32)
    seq_hr = _sc_gather(jnp.stack([gh, r]), idx3, B * L, D)
    h_seq = seq_hr[0]
    r_seq = seq_hr[1]

    bs = 8
    grid_b = B // bs
    seqblk = pl.BlockSpec((bs * L, D), lambda i: (i, 0))
    for l in range(NUM_GT):
        h_seq = pl.pallas_call(
            functools.partial(_agt_body, bs=bs),
            grid=(grid_b,),
            in_specs=[seqblk, seqblk, full((D, D)), full((D, D)),
                      full((1, HEAD_DIM)), full((1, HEAD_DIM)),
                      full((D, RL_DIM * NHEADS)), full((D, RL_DIM * NHEADS)),
                      full((D, D)), full((1, D)), full((1, D))],
            out_specs=seqblk,
            out_shape=jax.ShapeDtypeStruct((B * L, D), f32),
        )(h_seq, r_seq, gt_Wl[l], gt_Wr[l], gt_al[l].reshape(1, HEAD_DIM),
          gt_ar[l].reshape(1, HEAD_DIM), gt_Wrs[l], gt_Wrt[l], gt_Wf[l],
          gt_ln_g[l].reshape(1, D), gt_ln_b[l].reshape(1, D))

    x0 = h_seq.reshape(B, L, D)[:, 0, :]
    out = pl.pallas_call(
        _pred_body,
        in_specs=[pl.BlockSpec((B, D), lambda: (0, 0)),
                  pl.BlockSpec((D, C), lambda: (0, 0)),
                  pl.BlockSpec((1, C), lambda: (0, 0))],
        out_specs=pl.BlockSpec((B, C), lambda: (0, 0)),
        out_shape=jax.ShapeDtypeStruct((B, C), f32),
    )(x0, pred_W, pred_b.reshape(1, C))
    return out


# speedup vs baseline: 4.5178x; 1.9958x over previous
import functools

import jax
import jax.numpy as jnp
from jax import lax
from jax.experimental import pallas as pl
from jax.experimental.pallas import tpu as pltpu
from jax.experimental.pallas import tpu_sc as plsc

N = 10000
E = 320000
D = 128
T = 4
NHEADS = 4
HEAD_DIM = 32
RL_DIM = 128
NUM_GNNS = 4
NUM_GT = 2
B = 1024
L = 16
C = 7

NS = 16
KE = 80
BN = 1000


def _sc_segsum(table, gidx5, sidx5, zeros, np_rows, dt):
    nch = gidx5.shape[2]
    cb = gidx5.shape[3]
    rpt = np_rows // NS

    mesh = plsc.VectorSubcoreMesh(core_axis_name="c", subcore_axis_name="s")

    @functools.partial(
        pl.kernel,
        mesh=mesh,
        out_type=jax.ShapeDtypeStruct((2, np_rows, dt), jnp.float32),
        scratch_types=[
            pltpu.VMEM_SHARED((np_rows, dt), jnp.float32),
            pltpu.VMEM((cb, KE), jnp.int32),
            pltpu.VMEM((cb, KE), jnp.int32),
            pltpu.VMEM((KE, dt), jnp.float32),
            pltpu.VMEM((KE, dt), jnp.float32),
            pltpu.SemaphoreType.DMA,
            pltpu.SemaphoreType.DMA,
        ],
    )
    def k(table_h, gidx_h, sidx_h, zeros_h, out_h, acc, gi, si,
          rows0, rows1, sem0, sem1):
        c = lax.axis_index("c")
        s = lax.axis_index("s")
        rbuf = (rows0, rows1)
        sems = (sem0, sem1)
        pltpu.sync_copy(zeros_h.at[pl.ds(s * rpt, rpt)],
                        acc.at[pl.ds(s * rpt, rpt)])
        plsc.subcore_barrier()

        def chunk(t, carry):
            pltpu.sync_copy(gidx_h.at[c, s, t], gi)
            pltpu.sync_copy(sidx_h.at[c, s, t], si)
            pend = pltpu.async_copy(table_h.at[c].at[gi.at[0]], rbuf[0],
                                    sems[0])
            for j in range(cb):
                nxt = None
                if j + 1 < cb:
                    nxt = pltpu.async_copy(table_h.at[c].at[gi.at[j + 1]],
                                           rbuf[(j + 1) % 2],
                                           sems[(j + 1) % 2])
                pend.wait()
                pltpu.sync_copy(rbuf[j % 2], acc.at[si.at[j]], add=True)
                pend = nxt
            return carry

        lax.fori_loop(0, nch, chunk, 0)
        plsc.subcore_barrier()
        pltpu.sync_copy(acc.at[pl.ds(s * rpt, rpt)],
                        out_h.at[c, pl.ds(s * rpt, rpt)])

    return k(table, gidx5, sidx5, zeros)


def _sc_gather(table, idx3, n_out, dt):
    nbt = idx3.shape[1]
    kb = idx3.shape[2]

    mesh = plsc.VectorSubcoreMesh(core_axis_name="c", subcore_axis_name="s")

    @functools.partial(
        pl.kernel,
        mesh=mesh,
        out_type=jax.ShapeDtypeStruct((2, n_out, dt), jnp.float32),
        scratch_types=[
            pltpu.VMEM((nbt, kb), jnp.int32),
            pltpu.VMEM((kb, dt), jnp.float32),
            pltpu.SemaphoreType.DMA,
        ],
    )
    def k(table_h, idx_h, out_h, gi, rows, sem):
        c = lax.axis_index("c")
        s = lax.axis_index("s")
        pltpu.sync_copy(idx_h.at[s], gi)

        def body(j, carry):
            pltpu.async_copy(table_h.at[c].at[gi.at[j]], rows, sem).wait()
            pltpu.sync_copy(rows, out_h.at[c, pl.ds((s * nbt + j) * kb, kb)])
            return carry

        lax.fori_loop(0, nbt, body, 0)

    return k(table, idx3)


def _init_body(nt_ref, te_ref, hw_ref, hb_ref, pw_ref, pb_ref, rwt_ref,
               q_ref, wt_ref):
    nt = nt_ref[...]
    r0 = nt @ te_ref[...]
    z = jnp.maximum(r0 @ hw_ref[...] + hb_ref[...], 0.0)
    sc = z @ pw_ref[...] + pb_ref[...]
    sc = sc - jnp.max(sc, axis=-1, keepdims=True)
    ex = jnp.exp(sc)
    p = ex / jnp.sum(ex, axis=-1, keepdims=True)
    q_ref[:, 0:T] = p
    q_ref[:, T:T + 1] = jnp.ones((BN, 1), jnp.float32)
    q_ref[:, T + 1:] = jnp.zeros((BN, D - T - 1), jnp.float32)
    wt_ref[...] = nt @ rwt_ref[...]


def _proj_body(x_ref, w_ref, b_ref, o_ref):
    o_ref[0] = x_ref[0] @ w_ref[0] + b_ref[0]


def _postdeg_body(deg_ref, hfw_ref, hfb_ref, gh_ref, rew_ref, wt_ref,
                  iso_ref, isi_ref, r_ref, tab_ref):
    d0 = deg_ref[0]
    d1 = deg_ref[1]
    in_deg = jnp.maximum(d0[:, T:T + 1], 1.0)
    out_deg = jnp.maximum(d1[:, T:T + 1], 1.0)
    isi = lax.rsqrt(in_deg)
    iso = lax.rsqrt(out_deg)
    isi_ref[...] = isi
    iso_ref[...] = iso
    n_dist = d0[:, 0:T] / in_deg
    r = jnp.maximum(n_dist @ hfw_ref[...] + hfb_ref[...], 0.0)
    r_ref[...] = r
    tab_ref[0] = gh_ref[...] * iso
    tab_ref[1] = ((r * iso) @ rew_ref[...]) * wt_ref[...]


def _layer_body(agg_ref, iso_ref, isi_ref, gw_ref, gb_ref, rb_ref,
                rewn_ref, wtn_ref, gh_ref, r_ref, tab_ref):
    isi = isi_ref[...]
    iso = iso_ref[...]
    gh = jnp.maximum((agg_ref[0] @ gw_ref[...]) * isi + gb_ref[...], 0.0)
    r = jnp.maximum(agg_ref[1] * isi + rb_ref[...], 0.0)
    gh_ref[...] = gh
    r_ref[...] = r
    tab_ref[0] = gh * iso
    tab_ref[1] = ((r * iso) @ rewn_ref[...]) * wtn_ref[...]


def _agt_body(h_ref, rh_ref, wl_ref, wr_ref, al_ref, ar_ref, wrs_ref,
              wrt_ref, wf_ref, g_ref, b_ref, o_ref, bs):
    x = h_ref[...]
    rh = rh_ref[...]
    fl = x @ wl_ref[...]
    fr = x @ wr_ref[...]
    rk = rh @ wrs_ref[...]
    rq = rh @ wrt_ref[...]
    flk = jnp.where(fl > 0, fl, 0.01 * fl)
    frk = jnp.where(fr > 0, fr, 0.01 * fr)
    al = al_ref[...]
    ar = ar_ref[...]
    dn_rc = (((1,), (1,)), ((), ()))
    bl = bs * L
    ri = lax.broadcasted_iota(jnp.int32, (bl, bl), 0) // L
    ci = lax.broadcasted_iota(jnp.int32, (bl, bl), 1) // L
    neg = jnp.where(ri == ci, 0.0, -1e30)
    col_blocks = []
    for h in range(NHEADS):
        hc = slice(h * HEAD_DIM, (h + 1) * HEAD_DIM)
        rc = slice(h * RL_DIM, (h + 1) * RL_DIM)
        sl = lax.dot_general(flk[:, hc], al, dn_rc)
        sr = lax.dot_general(ar, frk[:, hc], dn_rc)
        s2 = lax.dot_general(rk[:, rc], rq[:, rc], dn_rc)
        sc = sl + sr + s2 + neg
        sc = sc - jnp.max(sc, axis=-1, keepdims=True)
        ex = jnp.exp(sc)
        sm = ex / jnp.sum(ex, axis=-1, keepdims=True)
        col_blocks.append(sm @ fr[:, hc])
    ctx = jnp.concatenate(col_blocks, axis=1)
    xo = x + ctx @ wf_ref[...]
    mu = jnp.mean(xo, axis=-1, keepdims=True)
    xc = xo - mu
    var = jnp.mean(xc * xc, axis=-1, keepdims=True)
    o_ref[...] = xc * lax.rsqrt(var + 1e-5) * g_ref[...] + b_ref[...]


def _pred_body(x_ref, w_ref, b_ref, o_ref):
    o_ref[...] = x_ref[...] @ w_ref[...] + b_ref[...]


def kernel(features_list, seqs, type_emb, node_type, edge_index, fc_W, fc_b,
           hade_W, hade_b, proto_W, proto_b, hfin_W, hfin_b, gcn_W, gcn_b,
           re_W, re_b, re_wt, gt_Wl, gt_Wr, gt_al, gt_ar, gt_Wrs, gt_Wrt,
           gt_Wf, gt_ln_g, gt_ln_b, pred_W, pred_b):
    f32 = jnp.float32
    src = edge_index[0].astype(jnp.int32)
    dst = edge_index[1].astype(jnp.int32)
    NP = 10240
    src3 = src.reshape(NS, 10, E // NS // KE // 10, KE)
    dst3 = dst.reshape(NS, 10, E // NS // KE // 10, KE)
    gidx = jnp.stack([src3, src3])
    sidx_dd = jnp.stack([dst3, dst3])
    sidx_ds = jnp.stack([dst3, src3])
    zeros_d = jnp.zeros((NP, D), f32)
    nt1h = jax.nn.one_hot(node_type, T, dtype=f32)

    grid_n = N // BN
    full = lambda shp: pl.BlockSpec(shp, lambda i: tuple(0 for _ in shp))
    rowblk = lambda w: pl.BlockSpec((BN, w), lambda i: (i, 0))
    rowblk2 = lambda w: pl.BlockSpec((2, BN, w), lambda i: (0, i, 0))

    q_tab, wtmap = pl.pallas_call(
        _init_body,
        grid=(grid_n,),
        in_specs=[rowblk(T), full((T, T)), full((T, D)), full((1, D)),
                  full((D, T)), full((1, T)), full((T, NUM_GNNS))],
        out_specs=[rowblk(D), rowblk(NUM_GNNS)],
        out_shape=[jax.ShapeDtypeStruct((N, D), f32),
                   jax.ShapeDtypeStruct((N, NUM_GNNS), f32)],
    )(nt1h, type_emb, hade_W, hade_b.reshape(1, D), proto_W,
      proto_b.reshape(1, T), re_wt.T)

    gh0 = pl.pallas_call(
        _proj_body,
        grid=(T,),
        in_specs=[pl.BlockSpec((1, N // T, D), lambda t: (t, 0, 0)),
                  pl.BlockSpec((1, D, D), lambda t: (t, 0, 0)),
                  pl.BlockSpec((1, 1, D), lambda t: (t, 0, 0))],
        out_specs=pl.BlockSpec((1, N // T, D), lambda t: (t, 0, 0)),
        out_shape=jax.ShapeDtypeStruct((T, N // T, D), f32),
    )(features_list, fc_W, fc_b.reshape(T, 1, D)).reshape(N, D)

    deg = _sc_segsum(jnp.stack([q_tab, q_tab]), gidx, sidx_ds, zeros_d, NP, D)

    iso, isi, r, tab = pl.pallas_call(
        _postdeg_body,
        grid=(grid_n,),
        in_specs=[rowblk2(D), full((T, D)), full((1, D)), rowblk(D),
                  full((D, D)), rowblk(1)],
        out_specs=[rowblk(1), rowblk(1), rowblk(D), rowblk2(D)],
        out_shape=[jax.ShapeDtypeStruct((N, 1), f32),
                   jax.ShapeDtypeStruct((N, 1), f32),
                   jax.ShapeDtypeStruct((N, D), f32),
                   jax.ShapeDtypeStruct((2, N, D), f32)],
    )(deg, hfin_W, hfin_b.reshape(1, D), gh0, re_W[0], wtmap[:, 0:1])

    gh = gh0
    for l in range(NUM_GNNS):
        agg = _sc_segsum(tab, gidx, sidx_dd, zeros_d, NP, D)
        ln = min(l + 1, NUM_GNNS - 1)
        gh, r, tab = pl.pallas_call(
            _layer_body,
            grid=(grid_n,),
            in_specs=[rowblk2(D), rowblk(1), rowblk(1), full((D, D)),
                      full((1, D)), full((1, D)), full((D, D)), rowblk(1)],
            out_specs=[rowblk(D), rowblk(D), rowblk2(D)],
            out_shape=[jax.ShapeDtypeStruct((N, D), f32),
                       jax.ShapeDtypeStruct((N, D), f32),
                       jax.ShapeDtypeStruct((2, N, D), f32)],
        )(agg, iso, isi, gcn_W[l], gcn_b[l].reshape(1, D),
          re_b[l].reshape(1, D), re_W[ln], wtmap[:, ln:ln + 1])

    kb = 64
    idx3 = seqs.reshape(NS, B * L // kb // NS, kb).astype(jnp.int32)
    seq_hr = _sc_gather(jnp.stack([gh, r]), idx3, B * L, D)
    h_seq = seq_hr[0]
    r_seq = seq_hr[1]

    bs = 8
    grid_b = B // bs
    seqblk = pl.BlockSpec((bs * L, D), lambda i: (i, 0))
    for l in range(NUM_GT):
        h_seq = pl.pallas_call(
            functools.partial(_agt_body, bs=bs),
            grid=(grid_b,),
            in_specs=[seqblk, seqblk, full((D, D)), full((D, D)),
                      full((1, HEAD_DIM)), full((1, HEAD_DIM)),
                      full((D, RL_DIM * NHEADS)), full((D, RL_DIM * NHEADS)),
                      full((D, D)), full((1, D)), full((1, D))],
            out_specs=seqblk,
            out_shape=jax.ShapeDtypeStruct((B * L, D), f32),
        )(h_seq, r_seq, gt_Wl[l], gt_Wr[l], gt_al[l].reshape(1, HEAD_DIM),
          gt_ar[l].reshape(1, HEAD_DIM), gt_Wrs[l], gt_Wrt[l], gt_Wf[l],
          gt_ln_g[l].reshape(1, D), gt_ln_b[l].reshape(1, D))

    x0 = h_seq.reshape(B, L, D)[:, 0, :]
    out = pl.pallas_call(
        _pred_body,
        in_specs=[pl.BlockSpec((B, D), lambda: (0, 0)),
                  pl.BlockSpec((D, C), lambda: (0, 0)),
                  pl.BlockSpec((1, C), lambda: (0, 0))],
        out_specs=pl.BlockSpec((B, C), lambda: (0, 0)),
        out_shape=jax.ShapeDtypeStruct((B, C), f32),
    )(x0, pred_W, pred_b.reshape(1, C))
    return out


# speedup vs baseline: 5.0326x; 1.1139x over previous
import functools

import jax
import jax.numpy as jnp
from jax import lax
from jax.experimental import pallas as pl
from jax.experimental.pallas import tpu as pltpu
from jax.experimental.pallas import tpu_sc as plsc

N = 10000
E = 320000
D = 128
T = 4
NHEADS = 4
HEAD_DIM = 32
RL_DIM = 128
NUM_GNNS = 4
NUM_GT = 2
B = 1024
L = 16
C = 7

NS = 16
KE = 80
BN = 1000


def _sc_segsum(table, gidx5, sidx5, zeros, np_rows, dt):
    nch = gidx5.shape[2]
    cb = gidx5.shape[3]
    rpt = np_rows // NS

    mesh = plsc.VectorSubcoreMesh(core_axis_name="c", subcore_axis_name="s")

    @functools.partial(
        pl.kernel,
        mesh=mesh,
        out_type=jax.ShapeDtypeStruct((2, np_rows, dt), jnp.float32),
        scratch_types=[
            pltpu.VMEM_SHARED((np_rows, dt), jnp.float32),
            pltpu.VMEM((cb, KE), jnp.int32),
            pltpu.VMEM((cb, KE), jnp.int32),
            pltpu.VMEM((KE, dt), jnp.float32),
            pltpu.VMEM((KE, dt), jnp.float32),
            pltpu.VMEM((KE, dt), jnp.float32),
            pltpu.SemaphoreType.DMA,
            pltpu.SemaphoreType.DMA,
            pltpu.SemaphoreType.DMA,
            pltpu.SemaphoreType.DMA,
            pltpu.SemaphoreType.DMA,
            pltpu.SemaphoreType.DMA,
        ],
    )
    def k(table_h, gidx_h, sidx_h, zeros_h, out_h, acc, gi, si,
          rows0, rows1, rows2, g0, g1, g2, s0, s1, s2):
        c = lax.axis_index("c")
        s = lax.axis_index("s")
        rbuf = (rows0, rows1, rows2)
        gsem = (g0, g1, g2)
        ssem = (s0, s1, s2)
        pltpu.sync_copy(zeros_h.at[pl.ds(s * rpt, rpt)],
                        acc.at[pl.ds(s * rpt, rpt)])
        plsc.subcore_barrier()

        def chunk(t, carry):
            pltpu.sync_copy(gidx_h.at[c, s, t], gi)
            pltpu.sync_copy(sidx_h.at[c, s, t], si)
            pend_g = [None, None, None]
            pend_s = [None, None, None]
            for j in range(min(2, cb)):
                pend_g[j % 3] = pltpu.async_copy(
                    table_h.at[c].at[gi.at[j]], rbuf[j % 3], gsem[j % 3])
            for j in range(cb):
                b = j % 3
                jn = j + 2
                if jn < cb:
                    bn = jn % 3
                    if pend_s[bn] is not None:
                        pend_s[bn].wait()
                        pend_s[bn] = None
                    pend_g[bn] = pltpu.async_copy(
                        table_h.at[c].at[gi.at[jn]], rbuf[bn], gsem[bn])
                pend_g[b].wait()
                pend_s[b] = pltpu.async_copy(rbuf[b], acc.at[si.at[j]],
                                             ssem[b], add=True)
            for b in range(3):
                if pend_s[b] is not None:
                    pend_s[b].wait()
            return carry

        lax.fori_loop(0, nch, chunk, 0)
        plsc.subcore_barrier()
        pltpu.sync_copy(acc.at[pl.ds(s * rpt, rpt)],
                        out_h.at[c, pl.ds(s * rpt, rpt)])

    return k(table, gidx5, sidx5, zeros)


def _sc_gather(table, idx3, n_out, dt):
    nbt = idx3.shape[1]
    kb = idx3.shape[2]

    mesh = plsc.VectorSubcoreMesh(core_axis_name="c", subcore_axis_name="s")

    @functools.partial(
        pl.kernel,
        mesh=mesh,
        out_type=jax.ShapeDtypeStruct((2, n_out, dt), jnp.float32),
        scratch_types=[
            pltpu.VMEM((nbt, kb), jnp.int32),
            pltpu.VMEM((kb, dt), jnp.float32),
            pltpu.SemaphoreType.DMA,
        ],
    )
    def k(table_h, idx_h, out_h, gi, rows, sem):
        c = lax.axis_index("c")
        s = lax.axis_index("s")
        pltpu.sync_copy(idx_h.at[s], gi)

        def body(j, carry):
            pltpu.async_copy(table_h.at[c].at[gi.at[j]], rows, sem).wait()
            pltpu.sync_copy(rows, out_h.at[c, pl.ds((s * nbt + j) * kb, kb)])
            return carry

        lax.fori_loop(0, nbt, body, 0)

    return k(table, idx3)


def _init_body(nt_ref, te_ref, hw_ref, hb_ref, pw_ref, pb_ref, rwt_ref,
               q_ref, wt_ref):
    nt = nt_ref[...]
    r0 = nt @ te_ref[...]
    z = jnp.maximum(r0 @ hw_ref[...] + hb_ref[...], 0.0)
    sc = z @ pw_ref[...] + pb_ref[...]
    sc = sc - jnp.max(sc, axis=-1, keepdims=True)
    ex = jnp.exp(sc)
    p = ex / jnp.sum(ex, axis=-1, keepdims=True)
    q_ref[:, 0:T] = p
    q_ref[:, T:T + 1] = jnp.ones((BN, 1), jnp.float32)
    q_ref[:, T + 1:] = jnp.zeros((BN, D - T - 1), jnp.float32)
    wt_ref[...] = nt @ rwt_ref[...]


def _proj_body(x_ref, w_ref, b_ref, o_ref):
    o_ref[0] = x_ref[0] @ w_ref[0] + b_ref[0]


def _postdeg_body(deg_ref, hfw_ref, hfb_ref, gh_ref, rew_ref, wt_ref,
                  iso_ref, isi_ref, r_ref, tab_ref):
    d0 = deg_ref[0]
    d1 = deg_ref[1]
    in_deg = jnp.maximum(d0[:, T:T + 1], 1.0)
    out_deg = jnp.maximum(d1[:, T:T + 1], 1.0)
    isi = lax.rsqrt(in_deg)
    iso = lax.rsqrt(out_deg)
    isi_ref[...] = isi
    iso_ref[...] = iso
    n_dist = d0[:, 0:T] / in_deg
    r = jnp.maximum(n_dist @ hfw_ref[...] + hfb_ref[...], 0.0)
    r_ref[...] = r
    tab_ref[0] = gh_ref[...] * iso
    tab_ref[1] = ((r * iso) @ rew_ref[...]) * wt_ref[...]


def _layer_body(agg_ref, iso_ref, isi_ref, gw_ref, gb_ref, rb_ref,
                rewn_ref, wtn_ref, gh_ref, r_ref, tab_ref):
    isi = isi_ref[...]
    iso = iso_ref[...]
    gh = jnp.maximum((agg_ref[0] @ gw_ref[...]) * isi + gb_ref[...], 0.0)
    r = jnp.maximum(agg_ref[1] * isi + rb_ref[...], 0.0)
    gh_ref[...] = gh
    r_ref[...] = r
    tab_ref[0] = gh * iso
    tab_ref[1] = ((r * iso) @ rewn_ref[...]) * wtn_ref[...]


def _agt_body(h_ref, rh_ref, wl_ref, wr_ref, al_ref, ar_ref, wrs_ref,
              wrt_ref, wf_ref, g_ref, b_ref, o_ref, bs):
    x = h_ref[...]
    rh = rh_ref[...]
    fl = x @ wl_ref[...]
    fr = x @ wr_ref[...]
    rk = rh @ wrs_ref[...]
    rq = rh @ wrt_ref[...]
    flk = jnp.where(fl > 0, fl, 0.01 * fl)
    frk = jnp.where(fr > 0, fr, 0.01 * fr)
    al = al_ref[...]
    ar = ar_ref[...]
    dn_rc = (((1,), (1,)), ((), ()))
    bl = bs * L
    ri = lax.broadcasted_iota(jnp.int32, (bl, bl), 0) // L
    ci = lax.broadcasted_iota(jnp.int32, (bl, bl), 1) // L
    neg = jnp.where(ri == ci, 0.0, -1e30)
    col_blocks = []
    for h in range(NHEADS):
        hc = slice(h * HEAD_DIM, (h + 1) * HEAD_DIM)
        rc = slice(h * RL_DIM, (h + 1) * RL_DIM)
        sl = lax.dot_general(flk[:, hc], al, dn_rc)
        sr = lax.dot_general(ar, frk[:, hc], dn_rc)
        s2 = lax.dot_general(rk[:, rc], rq[:, rc], dn_rc)
        sc = sl + sr + s2 + neg
        sc = sc - jnp.max(sc, axis=-1, keepdims=True)
        ex = jnp.exp(sc)
        sm = ex / jnp.sum(ex, axis=-1, keepdims=True)
        col_blocks.append(sm @ fr[:, hc])
    ctx = jnp.concatenate(col_blocks, axis=1)
    xo = x + ctx @ wf_ref[...]
    mu = jnp.mean(xo, axis=-1, keepdims=True)
    xc = xo - mu
    var = jnp.mean(xc * xc, axis=-1, keepdims=True)
    o_ref[...] = xc * lax.rsqrt(var + 1e-5) * g_ref[...] + b_ref[...]


def _pred_body(x_ref, w_ref, b_ref, o_ref):
    o_ref[...] = x_ref[...] @ w_ref[...] + b_ref[...]


def kernel(features_list, seqs, type_emb, node_type, edge_index, fc_W, fc_b,
           hade_W, hade_b, proto_W, proto_b, hfin_W, hfin_b, gcn_W, gcn_b,
           re_W, re_b, re_wt, gt_Wl, gt_Wr, gt_al, gt_ar, gt_Wrs, gt_Wrt,
           gt_Wf, gt_ln_g, gt_ln_b, pred_W, pred_b):
    f32 = jnp.float32
    src = edge_index[0].astype(jnp.int32)
    dst = edge_index[1].astype(jnp.int32)
    NP = 10240
    src3 = src.reshape(NS, 10, E // NS // KE // 10, KE)
    dst3 = dst.reshape(NS, 10, E // NS // KE // 10, KE)
    gidx = jnp.stack([src3, src3])
    sidx_dd = jnp.stack([dst3, dst3])
    sidx_ds = jnp.stack([dst3, src3])
    zeros_d = jnp.zeros((NP, D), f32)
    nt1h = jax.nn.one_hot(node_type, T, dtype=f32)

    grid_n = N // BN
    full = lambda shp: pl.BlockSpec(shp, lambda i: tuple(0 for _ in shp))
    rowblk = lambda w: pl.BlockSpec((BN, w), lambda i: (i, 0))
    rowblk2 = lambda w: pl.BlockSpec((2, BN, w), lambda i: (0, i, 0))

    q_tab, wtmap = pl.pallas_call(
        _init_body,
        grid=(grid_n,),
        in_specs=[rowblk(T), full((T, T)), full((T, D)), full((1, D)),
                  full((D, T)), full((1, T)), full((T, NUM_GNNS))],
        out_specs=[rowblk(D), rowblk(NUM_GNNS)],
        out_shape=[jax.ShapeDtypeStruct((N, D), f32),
                   jax.ShapeDtypeStruct((N, NUM_GNNS), f32)],
    )(nt1h, type_emb, hade_W, hade_b.reshape(1, D), proto_W,
      proto_b.reshape(1, T), re_wt.T)

    gh0 = pl.pallas_call(
        _proj_body,
        grid=(T,),
        in_specs=[pl.BlockSpec((1, N // T, D), lambda t: (t, 0, 0)),
                  pl.BlockSpec((1, D, D), lambda t: (t, 0, 0)),
                  pl.BlockSpec((1, 1, D), lambda t: (t, 0, 0))],
        out_specs=pl.BlockSpec((1, N // T, D), lambda t: (t, 0, 0)),
        out_shape=jax.ShapeDtypeStruct((T, N // T, D), f32),
    )(features_list, fc_W, fc_b.reshape(T, 1, D)).reshape(N, D)

    deg = _sc_segsum(jnp.stack([q_tab, q_tab]), gidx, sidx_ds, zeros_d, NP, D)

    iso, isi, r, tab = pl.pallas_call(
        _postdeg_body,
        grid=(grid_n,),
        in_specs=[rowblk2(D), full((T, D)), full((1, D)), rowblk(D),
                  full((D, D)), rowblk(1)],
        out_specs=[rowblk(1), rowblk(1), rowblk(D), rowblk2(D)],
        out_shape=[jax.ShapeDtypeStruct((N, 1), f32),
                   jax.ShapeDtypeStruct((N, 1), f32),
                   jax.ShapeDtypeStruct((N, D), f32),
                   jax.ShapeDtypeStruct((2, N, D), f32)],
    )(deg, hfin_W, hfin_b.reshape(1, D), gh0, re_W[0], wtmap[:, 0:1])

    gh = gh0
    for l in range(NUM_GNNS):
        agg = _sc_segsum(tab, gidx, sidx_dd, zeros_d, NP, D)
        ln = min(l + 1, NUM_GNNS - 1)
        gh, r, tab = pl.pallas_call(
            _layer_body,
            grid=(grid_n,),
            in_specs=[rowblk2(D), rowblk(1), rowblk(1), full((D, D)),
                      full((1, D)), full((1, D)), full((D, D)), rowblk(1)],
            out_specs=[rowblk(D), rowblk(D), rowblk2(D)],
            out_shape=[jax.ShapeDtypeStruct((N, D), f32),
                       jax.ShapeDtypeStruct((N, D), f32),
                       jax.ShapeDtypeStruct((2, N, D), f32)],
        )(agg, iso, isi, gcn_W[l], gcn_b[l].reshape(1, D),
          re_b[l].reshape(1, D), re_W[ln], wtmap[:, ln:ln + 1])

    kb = 64
    idx3 = seqs.reshape(NS, B * L // kb // NS, kb).astype(jnp.int32)
    seq_hr = _sc_gather(jnp.stack([gh, r]), idx3, B * L, D)
    h_seq = seq_hr[0]
    r_seq = seq_hr[1]

    bs = 8
    grid_b = B // bs
    seqblk = pl.BlockSpec((bs * L, D), lambda i: (i, 0))
    for l in range(NUM_GT):
        h_seq = pl.pallas_call(
            functools.partial(_agt_body, bs=bs),
            grid=(grid_b,),
            in_specs=[seqblk, seqblk, full((D, D)), full((D, D)),
                      full((1, HEAD_DIM)), full((1, HEAD_DIM)),
                      full((D, RL_DIM * NHEADS)), full((D, RL_DIM * NHEADS)),
                      full((D, D)), full((1, D)), full((1, D))],
            out_specs=seqblk,
            out_shape=jax.ShapeDtypeStruct((B * L, D), f32),
        )(h_seq, r_seq, gt_Wl[l], gt_Wr[l], gt_al[l].reshape(1, HEAD_DIM),
          gt_ar[l].reshape(1, HEAD_DIM), gt_Wrs[l], gt_Wrt[l], gt_Wf[l],
          gt_ln_g[l].reshape(1, D), gt_ln_b[l].reshape(1, D))

    x0 = h_seq.reshape(B, L, D)[:, 0, :]
    out = pl.pallas_call(
        _pred_body,
        in_specs=[pl.BlockSpec((B, D), lambda: (0, 0)),
                  pl.BlockSpec((D, C), lambda: (0, 0)),
                  pl.BlockSpec((1, C), lambda: (0, 0))],
        out_specs=pl.BlockSpec((B, C), lambda: (0, 0)),
        out_shape=jax.ShapeDtypeStruct((B, C), f32),
    )(x0, pred_W, pred_b.reshape(1, C))
    return out


# speedup vs baseline: 5.7401x; 1.1406x over previous
import functools

import jax
import jax.numpy as jnp
from jax import lax
from jax.experimental import pallas as pl
from jax.experimental.pallas import tpu as pltpu
from jax.experimental.pallas import tpu_sc as plsc

N = 10000
E = 320000
D = 128
T = 4
NHEADS = 4
HEAD_DIM = 32
RL_DIM = 128
NUM_GNNS = 4
NUM_GT = 2
B = 1024
L = 16
C = 7

NS = 16
KE = 80
BN = 1000


def _sc_segsum(table, gidx5, sidx5, zeros, np_rows, dt):
    nch = gidx5.shape[2]
    cb = gidx5.shape[3]
    rpt = np_rows // NS

    mesh = plsc.VectorSubcoreMesh(core_axis_name="c", subcore_axis_name="s")

    @functools.partial(
        pl.kernel,
        mesh=mesh,
        out_type=jax.ShapeDtypeStruct((2, np_rows, dt), jnp.float32),
        scratch_types=[
            pltpu.VMEM_SHARED((np_rows, dt), jnp.float32),
            pltpu.VMEM((cb, KE), jnp.int32),
            pltpu.VMEM((cb, KE), jnp.int32),
            pltpu.VMEM((KE, dt), jnp.float32),
            pltpu.VMEM((KE, dt), jnp.float32),
            pltpu.VMEM((KE, dt), jnp.float32),
            pltpu.SemaphoreType.DMA,
            pltpu.SemaphoreType.DMA,
            pltpu.SemaphoreType.DMA,
            pltpu.SemaphoreType.DMA,
            pltpu.SemaphoreType.DMA,
            pltpu.SemaphoreType.DMA,
        ],
    )
    def k(table_h, gidx_h, sidx_h, zeros_h, out_h, acc, gi, si,
          rows0, rows1, rows2, g0, g1, g2, s0, s1, s2):
        c = lax.axis_index("c")
        s = lax.axis_index("s")
        rbuf = (rows0, rows1, rows2)
        gsem = (g0, g1, g2)
        ssem = (s0, s1, s2)
        pltpu.sync_copy(zeros_h.at[pl.ds(s * rpt, rpt)],
                        acc.at[pl.ds(s * rpt, rpt)])
        plsc.subcore_barrier()

        def chunk(t, carry):
            pltpu.sync_copy(gidx_h.at[c, s, t], gi)
            pltpu.sync_copy(sidx_h.at[c, s, t], si)
            pend_g = [None, None, None]
            pend_s = [None, None, None]
            for j in range(min(2, cb)):
                pend_g[j % 3] = pltpu.async_copy(
                    table_h.at[c].at[gi.at[j]], rbuf[j % 3], gsem[j % 3])
            for j in range(cb):
                b = j % 3
                jn = j + 2
                if jn < cb:
                    bn = jn % 3
                    if pend_s[bn] is not None:
                        pend_s[bn].wait()
                        pend_s[bn] = None
                    pend_g[bn] = pltpu.async_copy(
                        table_h.at[c].at[gi.at[jn]], rbuf[bn], gsem[bn])
                pend_g[b].wait()
                pend_s[b] = pltpu.async_copy(rbuf[b], acc.at[si.at[j]],
                                             ssem[b], add=True)
            for b in range(3):
                if pend_s[b] is not None:
                    pend_s[b].wait()
            return carry

        lax.fori_loop(0, nch, chunk, 0)
        plsc.subcore_barrier()
        pltpu.sync_copy(acc.at[pl.ds(s * rpt, rpt)],
                        out_h.at[c, pl.ds(s * rpt, rpt)])

    return k(table, gidx5, sidx5, zeros)


def _sc_gather(table, idx3, n_out, dt):
    nbt = idx3.shape[1]
    kb = idx3.shape[2]

    mesh = plsc.VectorSubcoreMesh(core_axis_name="c", subcore_axis_name="s")

    @functools.partial(
        pl.kernel,
        mesh=mesh,
        out_type=jax.ShapeDtypeStruct((2, n_out, dt), jnp.float32),
        scratch_types=[
            pltpu.VMEM((nbt, kb), jnp.int32),
            pltpu.VMEM((kb, dt), jnp.float32),
            pltpu.SemaphoreType.DMA,
        ],
    )
    def k(table_h, idx_h, out_h, gi, rows, sem):
        c = lax.axis_index("c")
        s = lax.axis_index("s")
        pltpu.sync_copy(idx_h.at[s], gi)

        def body(j, carry):
            pltpu.async_copy(table_h.at[c].at[gi.at[j]], rows, sem).wait()
            pltpu.sync_copy(rows, out_h.at[c, pl.ds((s * nbt + j) * kb, kb)])
            return carry

        lax.fori_loop(0, nbt, body, 0)

    return k(table, idx3)


def _init_body(nt_ref, te_ref, hw_ref, hb_ref, pw_ref, pb_ref, rwt_ref,
               q_ref, wt_ref):
    nt = nt_ref[...]
    r0 = nt @ te_ref[...]
    z = jnp.maximum(r0 @ hw_ref[...] + hb_ref[...], 0.0)
    sc = z @ pw_ref[...] + pb_ref[...]
    sc = sc - jnp.max(sc, axis=-1, keepdims=True)
    ex = jnp.exp(sc)
    p = ex / jnp.sum(ex, axis=-1, keepdims=True)
    q_ref[:, 0:T] = p
    q_ref[:, T:T + 1] = jnp.ones((BN, 1), jnp.float32)
    q_ref[:, T + 1:] = jnp.zeros((BN, D - T - 1), jnp.float32)
    wt_ref[...] = nt @ rwt_ref[...]


def _proj_body(x_ref, w_ref, b_ref, o_ref):
    o_ref[0] = x_ref[0] @ w_ref[0] + b_ref[0]


def _postdeg_body(deg_ref, hfw_ref, hfb_ref, gh_ref, rew_ref, wt_ref,
                  iso_ref, isi_ref, r_ref, tab_ref):
    d0 = deg_ref[0]
    d1 = deg_ref[1]
    in_deg = jnp.maximum(d0[:, T:T + 1], 1.0)
    out_deg = jnp.maximum(d1[:, T:T + 1], 1.0)
    isi = lax.rsqrt(in_deg)
    iso = lax.rsqrt(out_deg)
    isi_ref[...] = isi
    iso_ref[...] = iso
    n_dist = d0[:, 0:T] / in_deg
    r = jnp.maximum(n_dist @ hfw_ref[...] + hfb_ref[...], 0.0)
    r_ref[...] = r
    tab_ref[0] = gh_ref[...] * iso
    tab_ref[1] = ((r * iso) @ rew_ref[...]) * wt_ref[...]


def _layer_body(agg_ref, iso_ref, isi_ref, gw_ref, gb_ref, rb_ref,
                rewn_ref, wtn_ref, gh_ref, r_ref, tab_ref):
    isi = isi_ref[...]
    iso = iso_ref[...]
    gh = jnp.maximum((agg_ref[0] @ gw_ref[...]) * isi + gb_ref[...], 0.0)
    r = jnp.maximum(agg_ref[1] * isi + rb_ref[...], 0.0)
    gh_ref[...] = gh
    r_ref[...] = r
    tab_ref[0] = gh * iso
    tab_ref[1] = ((r * iso) @ rewn_ref[...]) * wtn_ref[...]


def _agt_body(h_ref, rh_ref, wl_ref, wr_ref, al_ref, ar_ref, wrs_ref,
              wrt_ref, wf_ref, g_ref, b_ref, o_ref, bs):
    x = h_ref[...]
    rh = rh_ref[...]
    fl = x @ wl_ref[...]
    fr = x @ wr_ref[...]
    rk = rh @ wrs_ref[...]
    rq = rh @ wrt_ref[...]
    flk = jnp.where(fl > 0, fl, 0.01 * fl)
    frk = jnp.where(fr > 0, fr, 0.01 * fr)
    al = al_ref[...]
    ar = ar_ref[...]
    dn_rc = (((1,), (1,)), ((), ()))
    bl = bs * L
    ri = lax.broadcasted_iota(jnp.int32, (bl, bl), 0) // L
    ci = lax.broadcasted_iota(jnp.int32, (bl, bl), 1) // L
    neg = jnp.where(ri == ci, 0.0, -1e30)
    col_blocks = []
    for h in range(NHEADS):
        hc = slice(h * HEAD_DIM, (h + 1) * HEAD_DIM)
        rc = slice(h * RL_DIM, (h + 1) * RL_DIM)
        sl = lax.dot_general(flk[:, hc], al, dn_rc)
        sr = lax.dot_general(ar, frk[:, hc], dn_rc)
        s2 = lax.dot_general(rk[:, rc], rq[:, rc], dn_rc)
        sc = sl + sr + s2 + neg
        sc = sc - jnp.max(sc, axis=-1, keepdims=True)
        ex = jnp.exp(sc)
        sm = ex / jnp.sum(ex, axis=-1, keepdims=True)
        col_blocks.append(sm @ fr[:, hc])
    ctx = jnp.concatenate(col_blocks, axis=1)
    xo = x + ctx @ wf_ref[...]
    mu = jnp.mean(xo, axis=-1, keepdims=True)
    xc = xo - mu
    var = jnp.mean(xc * xc, axis=-1, keepdims=True)
    o_ref[...] = xc * lax.rsqrt(var + 1e-5) * g_ref[...] + b_ref[...]


def _pred_body(x_ref, w_ref, b_ref, o_ref):
    o_ref[...] = x_ref[...] @ w_ref[...] + b_ref[...]


def kernel(features_list, seqs, type_emb, node_type, edge_index, fc_W, fc_b,
           hade_W, hade_b, proto_W, proto_b, hfin_W, hfin_b, gcn_W, gcn_b,
           re_W, re_b, re_wt, gt_Wl, gt_Wr, gt_al, gt_ar, gt_Wrs, gt_Wrt,
           gt_Wf, gt_ln_g, gt_ln_b, pred_W, pred_b):
    f32 = jnp.float32
    src = edge_index[0].astype(jnp.int32)
    dst = edge_index[1].astype(jnp.int32)
    NP = 10240
    src3 = src.reshape(NS, 10, E // NS // KE // 10, KE)
    dst3 = dst.reshape(NS, 10, E // NS // KE // 10, KE)
    gidx = jnp.stack([src3, src3])
    sidx_dd = jnp.stack([dst3, dst3])
    sidx_ds = jnp.stack([dst3, src3])
    zeros_d = jnp.zeros((NP, D), f32)
    nt1h = jax.nn.one_hot(node_type, T, dtype=f32)

    grid_n = N // BN
    full = lambda shp: pl.BlockSpec(shp, lambda i: tuple(0 for _ in shp))
    rowblk = lambda w: pl.BlockSpec((BN, w), lambda i: (i, 0))
    rowblk2 = lambda w: pl.BlockSpec((2, BN, w), lambda i: (0, i, 0))

    q_tab, wtmap = pl.pallas_call(
        _init_body,
        grid=(grid_n,),
        in_specs=[rowblk(T), full((T, T)), full((T, D)), full((1, D)),
                  full((D, T)), full((1, T)), full((T, NUM_GNNS))],
        out_specs=[rowblk(D), rowblk(NUM_GNNS)],
        out_shape=[jax.ShapeDtypeStruct((N, D), f32),
                   jax.ShapeDtypeStruct((N, NUM_GNNS), f32)],
    )(nt1h, type_emb, hade_W, hade_b.reshape(1, D), proto_W,
      proto_b.reshape(1, T), re_wt.T)

    gh0 = pl.pallas_call(
        _proj_body,
        grid=(T,),
        in_specs=[pl.BlockSpec((1, N // T, D), lambda t: (t, 0, 0)),
                  pl.BlockSpec((1, D, D), lambda t: (t, 0, 0)),
                  pl.BlockSpec((1, 1, D), lambda t: (t, 0, 0))],
        out_specs=pl.BlockSpec((1, N // T, D), lambda t: (t, 0, 0)),
        out_shape=jax.ShapeDtypeStruct((T, N // T, D), f32),
    )(features_list, fc_W, fc_b.reshape(T, 1, D)).reshape(N, D)

    deg = _sc_segsum(jnp.stack([q_tab, q_tab]), gidx, sidx_ds, zeros_d, NP, D)

    iso, isi, r, tab = pl.pallas_call(
        _postdeg_body,
        grid=(grid_n,),
        in_specs=[rowblk2(D), full((T, D)), full((1, D)), rowblk(D),
                  full((D, D)), rowblk(1)],
        out_specs=[rowblk(1), rowblk(1), rowblk(D), rowblk2(D)],
        out_shape=[jax.ShapeDtypeStruct((N, 1), f32),
                   jax.ShapeDtypeStruct((N, 1), f32),
                   jax.ShapeDtypeStruct((N, D), f32),
                   jax.ShapeDtypeStruct((2, N, D), f32)],
    )(deg, hfin_W, hfin_b.reshape(1, D), gh0, re_W[0], wtmap[:, 0:1])

    gh = gh0
    for l in range(NUM_GNNS):
        agg = _sc_segsum(tab, gidx, sidx_dd, zeros_d, NP, D)
        ln = min(l + 1, NUM_GNNS - 1)
        gh, r, tab = pl.pallas_call(
            _layer_body,
            grid=(grid_n,),
            in_specs=[rowblk2(D), rowblk(1), rowblk(1), full((D, D)),
                      full((1, D)), full((1, D)), full((D, D)), rowblk(1)],
            out_specs=[rowblk(D), rowblk(D), rowblk2(D)],
            out_shape=[jax.ShapeDtypeStruct((N, D), f32),
                       jax.ShapeDtypeStruct((N, D), f32),
                       jax.ShapeDtypeStruct((2, N, D), f32)],
        )(agg, iso, isi, gcn_W[l], gcn_b[l].reshape(1, D),
          re_b[l].reshape(1, D), re_W[ln], wtmap[:, ln:ln + 1])

    kb = 64
    idx3 = seqs.reshape(NS, B * L // kb // NS, kb).astype(jnp.int32)
    seq_hr = _sc_gather(jnp.stack([gh, r]), idx3, B * L, D)
    h_seq = seq_hr[0]
    r_seq = seq_hr[1]

    bs = 16
    grid_b = B // bs
    seqblk = pl.BlockSpec((bs * L, D), lambda i: (i, 0))
    for l in range(NUM_GT):
        h_seq = pl.pallas_call(
            functools.partial(_agt_body, bs=bs),
            grid=(grid_b,),
            in_specs=[seqblk, seqblk, full((D, D)), full((D, D)),
                      full((1, HEAD_DIM)), full((1, HEAD_DIM)),
                      full((D, RL_DIM * NHEADS)), full((D, RL_DIM * NHEADS)),
                      full((D, D)), full((1, D)), full((1, D))],
            out_specs=seqblk,
            out_shape=jax.ShapeDtypeStruct((B * L, D), f32),
        )(h_seq, r_seq, gt_Wl[l], gt_Wr[l], gt_al[l].reshape(1, HEAD_DIM),
          gt_ar[l].reshape(1, HEAD_DIM), gt_Wrs[l], gt_Wrt[l], gt_Wf[l],
          gt_ln_g[l].reshape(1, D), gt_ln_b[l].reshape(1, D))

    x0 = h_seq.reshape(B, L, D)[:, 0, :]
    out = pl.pallas_call(
        _pred_body,
        in_specs=[pl.BlockSpec((B, D), lambda: (0, 0)),
                  pl.BlockSpec((D, C), lambda: (0, 0)),
                  pl.BlockSpec((1, C), lambda: (0, 0))],
        out_specs=pl.BlockSpec((B, C), lambda: (0, 0)),
        out_shape=jax.ShapeDtypeStruct((B, C), f32),
    )(x0, pred_W, pred_b.reshape(1, C))
    return out
